# baseline (speedup 1.0000x reference)
import functools

import jax
import jax.numpy as jnp
from jax import lax
from jax.experimental import pallas as pl
from jax.experimental.pallas import tpu as pltpu
from jax.experimental.pallas import tpu_sc as plsc

N = 10000
E = 320000
D = 128

NC = 2
NS = 16
NW = NC * NS

K = 128
SB_ = 8
EPW = -(-E // NW)
EPW_PAD = -(-EPW // (2 * SB_ * K)) * (2 * SB_ * K)
CHUNKS = EPW_PAD // K
E_PAD = EPW_PAD * NW

ROWS_PER_TILE = 632
ACC_ROWS = NS * ROWS_PER_TILE

SB = SB_
NPAIR = CHUNKS // (2 * SB)
IDX_CHUNKS = CHUNKS + SB


def _mm_body(h_ref, w_ref, norm_ref, o_ref):
    o_ref[...] = (
        jnp.dot(h_ref[...], w_ref[...], preferred_element_type=jnp.float32)
        * norm_ref[...]
    )


def _epilogue_body(p_ref, norm_ref, bias_ref, t_ref, o_ref):
    s = p_ref[0] + p_ref[1]
    o_ref[...] = jnp.maximum(s * norm_ref[...] * t_ref[0, 0] + bias_ref[...], 0.0)


def _scatter_body(hw, srcr, dstr, zeros, out,
                  sA, dA, sB, dB, rows_v, acc, isem, gsem):
    cid = lax.axis_index("c")
    sid = lax.axis_index("s")
    wid = cid * NS + sid

    r0 = sid * ROWS_PER_TILE
    pltpu.sync_copy(zeros, acc.at[pl.ds(r0, ROWS_PER_TILE)])

    def idx_start(base, s_v, d_v, p):
        pltpu.async_copy(srcr.at[wid, pl.ds(base, SB)], s_v, isem.at[2 * p])
        pltpu.async_copy(dstr.at[wid, pl.ds(base, SB)], d_v, isem.at[2 * p + 1])

    def idx_wait(base, s_v, d_v, p):
        pltpu.make_async_copy(srcr.at[wid, pl.ds(base, SB)], s_v,
                              isem.at[2 * p]).wait()
        pltpu.make_async_copy(dstr.at[wid, pl.ds(base, SB)], d_v,
                              isem.at[2 * p + 1]).wait()

    def gather_start(s_v, c, b):
        pltpu.async_copy(hw.at[s_v.at[c]], rows_v.at[b], gsem.at[b])

    def gather_wait(s_v, c, b):
        pltpu.make_async_copy(hw.at[s_v.at[c]], rows_v.at[b],
                              gsem.at[b]).wait()

    def phase(s_v, d_v):
        gather_start(s_v, 0, 0)
        gather_start(s_v, 1, 1)
        for c in range(SB):
            b = c % 2
            gather_wait(s_v, c, b)
            pltpu.sync_copy(rows_v.at[b], acc.at[d_v.at[c]], add=True)
            if c + 2 < SB:
                gather_start(s_v, c + 2, b)

    idx_start(0, sA, dA, 0)
    plsc.subcore_barrier()

    def _pair(u, carry):
        base = pl.multiple_of(u * 2 * SB, SB)
        base_b = pl.multiple_of(base + SB, SB)
        base_a2 = pl.multiple_of(base + 2 * SB, SB)
        idx_start(base_b, sB, dB, 1)
        idx_wait(base, sA, dA, 0)
        phase(sA, dA)
        idx_start(base_a2, sA, dA, 0)
        idx_wait(base_b, sB, dB, 1)
        phase(sB, dB)
        return carry

    lax.fori_loop(0, NPAIR, _pair, 0)

    idx_wait(CHUNKS, sA, dA, 0)

    plsc.subcore_barrier()

    pltpu.sync_copy(acc.at[pl.ds(r0, ROWS_PER_TILE)],
                    out.at[cid, pl.ds(r0, ROWS_PER_TILE)])


@functools.partial(jax.jit, static_argnums=())
def _scatter_call(hw, srcr, dstr, zeros):
    mesh = plsc.VectorSubcoreMesh(
        core_axis_name="c", subcore_axis_name="s", num_cores=NC, num_subcores=NS
    )
    return pl.kernel(
        _scatter_body,
        out_type=jax.ShapeDtypeStruct((NC, ACC_ROWS, D), jnp.float32),
        mesh=mesh,
        scratch_types=[
            pltpu.VMEM((SB, K), jnp.int32),
            pltpu.VMEM((SB, K), jnp.int32),
            pltpu.VMEM((SB, K), jnp.int32),
            pltpu.VMEM((SB, K), jnp.int32),
            pltpu.VMEM((2, K, D), jnp.float32),
            pltpu.VMEM_SHARED((ACC_ROWS, D), jnp.float32),
            pltpu.SemaphoreType.DMA((4,)),
            pltpu.SemaphoreType.DMA((2,)),
        ],
    )(hw, srcr, dstr, zeros)


def kernel(t, h, edge_index, norm, weight, bias):
    hw = pl.pallas_call(
        _mm_body,
        out_shape=jax.ShapeDtypeStruct((N, D), jnp.float32),
    )(h, weight, norm)

    src = edge_index[0]
    dst = edge_index[1]
    pad = E_PAD - E
    srcr = jnp.pad(src, (0, pad)).reshape(NW, CHUNKS, K)
    dstr = jnp.pad(dst, (0, pad), constant_values=N).reshape(NW, CHUNKS, K)
    srcr = jnp.pad(srcr, ((0, 0), (0, SB), (0, 0)))
    dstr = jnp.pad(dstr, ((0, 0), (0, SB), (0, 0)))
    zeros = jnp.zeros((ROWS_PER_TILE, D), jnp.float32)

    parts = _scatter_call(hw, srcr, dstr, zeros)[:, :N, :]

    return pl.pallas_call(
        _epilogue_body,
        out_shape=jax.ShapeDtypeStruct((N, D), jnp.float32),
    )(parts, norm, bias, t.reshape(1, 1))


# speedup vs baseline: 1.7267x; 1.7267x over previous
import functools

import jax
import jax.numpy as jnp
from jax import lax
from jax.experimental import pallas as pl
from jax.experimental.pallas import tpu as pltpu
from jax.experimental.pallas import tpu_sc as plsc

N = 10000
E = 320000
D = 128

NC = 2
NS = 16
NW = NC * NS

K = 80
EPW = -(-E // NW)
EPW_PAD = -(-EPW // (2 * K)) * (2 * K)
CHUNKS = EPW_PAD // K
E_PAD = EPW_PAD * NW

ROWS_PER_TILE = 632
ACC_ROWS = NS * ROWS_PER_TILE


def _mm_body(h_ref, w_ref, norm_ref, o_ref):
    o_ref[...] = (
        jnp.dot(h_ref[...], w_ref[...], preferred_element_type=jnp.float32)
        * norm_ref[...]
    )


def _epilogue_body(p_ref, norm_ref, bias_ref, t_ref, o_ref):
    s = p_ref[0] + p_ref[1]
    o_ref[...] = jnp.maximum(s * norm_ref[...] * t_ref[0, 0] + bias_ref[...], 0.0)


def _scatter_body(hw, srcr, dstr, zeros, out,
                  src_v, dst_v, rows_v, acc, gsem):
    cid = lax.axis_index("c")
    sid = lax.axis_index("s")
    wid = cid * NS + sid

    r0 = sid * ROWS_PER_TILE
    pltpu.sync_copy(zeros, acc.at[pl.ds(r0, ROWS_PER_TILE)])

    pltpu.sync_copy(srcr.at[wid], src_v)
    pltpu.sync_copy(dstr.at[wid], dst_v)
    plsc.subcore_barrier()

    def gather_start(j, b):
        off = pl.multiple_of(j * K, 8)
        pltpu.async_copy(hw.at[src_v.at[pl.ds(off, K)]], rows_v.at[b],
                         gsem.at[b])

    def gather_wait(j, b):
        off = pl.multiple_of(j * K, 8)
        pltpu.make_async_copy(hw.at[src_v.at[pl.ds(off, K)]], rows_v.at[b],
                              gsem.at[b]).wait()

    gather_start(0, 0)
    gather_start(1, 1)

    def _step(it, carry):
        j = it * 2
        for b in range(2):
            gather_wait(j + b, b)
            pltpu.sync_copy(rows_v.at[b], acc.at[dst_v.at[j + b]], add=True)
            gather_start(j + 2 + b, b)
        return carry

    lax.fori_loop(0, (CHUNKS - 2) // 2, _step, 0)

    for b in range(2):
        j = CHUNKS - 2 + b
        gather_wait(j, b)
        pltpu.sync_copy(rows_v.at[b], acc.at[dst_v.at[j]], add=True)

    plsc.subcore_barrier()

    pltpu.sync_copy(acc.at[pl.ds(r0, ROWS_PER_TILE)],
                    out.at[cid, pl.ds(r0, ROWS_PER_TILE)])


@functools.partial(jax.jit, static_argnums=())
def _scatter_call(hw, srcr, dstr, zeros):
    mesh = plsc.VectorSubcoreMesh(
        core_axis_name="c", subcore_axis_name="s", num_cores=NC, num_subcores=NS
    )
    return pl.kernel(
        _scatter_body,
        out_type=jax.ShapeDtypeStruct((NC, ACC_ROWS, D), jnp.float32),
        mesh=mesh,
        scratch_types=[
            pltpu.VMEM((EPW_PAD,), jnp.int32),
            pltpu.VMEM((CHUNKS, K), jnp.int32),
            pltpu.VMEM((2, K, D), jnp.float32),
            pltpu.VMEM_SHARED((ACC_ROWS, D), jnp.float32),
            pltpu.SemaphoreType.DMA((2,)),
        ],
    )(hw, srcr, dstr, zeros)


def kernel(t, h, edge_index, norm, weight, bias):
    hw = pl.pallas_call(
        _mm_body,
        out_shape=jax.ShapeDtypeStruct((N, D), jnp.float32),
    )(h, weight, norm)

    src = edge_index[0]
    dst = edge_index[1]
    pad = E_PAD - E
    srcr = jnp.pad(src, (0, pad)).reshape(NW, EPW_PAD)
    dstr = jnp.pad(dst, (0, pad), constant_values=N).reshape(NW, CHUNKS, K)
    zeros = jnp.zeros((ROWS_PER_TILE, D), jnp.float32)

    parts = _scatter_call(hw, srcr, dstr, zeros)[:, :N, :]

    return pl.pallas_call(
        _epilogue_body,
        out_shape=jax.ShapeDtypeStruct((N, D), jnp.float32),
    )(parts, norm, bias, t.reshape(1, 1))


# speedup vs baseline: 1.8682x; 1.0820x over previous
import functools

import jax
import jax.numpy as jnp
from jax import lax
from jax.experimental import pallas as pl
from jax.experimental.pallas import tpu as pltpu
from jax.experimental.pallas import tpu_sc as plsc

N = 10000
E = 320000
D = 128

NC = 2
NS = 16
NW = NC * NS

K = 112
EPW = -(-E // NW)
EPW_PAD = -(-EPW // (2 * K)) * (2 * K)
CHUNKS = EPW_PAD // K
E_PAD = EPW_PAD * NW

ROWS_PER_TILE = 632
ACC_ROWS = NS * ROWS_PER_TILE


def _mm_body(h_ref, w_ref, norm_ref, o_ref):
    o_ref[...] = (
        jnp.dot(h_ref[...], w_ref[...], preferred_element_type=jnp.float32)
        * norm_ref[...]
    )


def _epilogue_body(p_ref, norm_ref, bias_ref, t_ref, o_ref):
    s = p_ref[0] + p_ref[1]
    o_ref[...] = jnp.maximum(s * norm_ref[...] * t_ref[0, 0] + bias_ref[...], 0.0)


def _scatter_body(hw, srcr, dstr, zeros, out,
                  src_v, dst_v, rows_v, acc, gsem):
    cid = lax.axis_index("c")
    sid = lax.axis_index("s")
    wid = cid * NS + sid

    r0 = sid * ROWS_PER_TILE
    pltpu.sync_copy(zeros, acc.at[pl.ds(r0, ROWS_PER_TILE)])

    pltpu.sync_copy(srcr.at[wid], src_v)
    pltpu.sync_copy(dstr.at[wid], dst_v)
    plsc.subcore_barrier()

    def gather_start(j, b):
        off = pl.multiple_of(j * K, 8)
        pltpu.async_copy(hw.at[src_v.at[pl.ds(off, K)]], rows_v.at[b],
                         gsem.at[b])

    def gather_wait(j, b):
        off = pl.multiple_of(j * K, 8)
        pltpu.make_async_copy(hw.at[src_v.at[pl.ds(off, K)]], rows_v.at[b],
                              gsem.at[b]).wait()

    gather_start(0, 0)
    gather_start(1, 1)

    def _step(it, carry):
        j = it * 2
        for b in range(2):
            gather_wait(j + b, b)
            offd = pl.multiple_of((j + b) * K, 8)
            pltpu.sync_copy(rows_v.at[b], acc.at[dst_v.at[pl.ds(offd, K)]],
                            add=True)
            gather_start(j + 2 + b, b)
        return carry

    lax.fori_loop(0, (CHUNKS - 2) // 2, _step, 0)

    for b in range(2):
        j = CHUNKS - 2 + b
        gather_wait(j, b)
        offd = pl.multiple_of(j * K, 8)
        pltpu.sync_copy(rows_v.at[b], acc.at[dst_v.at[pl.ds(offd, K)]],
                        add=True)

    plsc.subcore_barrier()

    pltpu.sync_copy(acc.at[pl.ds(r0, ROWS_PER_TILE)],
                    out.at[cid, pl.ds(r0, ROWS_PER_TILE)])


@functools.partial(jax.jit, static_argnums=())
def _scatter_call(hw, srcr, dstr, zeros):
    mesh = plsc.VectorSubcoreMesh(
        core_axis_name="c", subcore_axis_name="s", num_cores=NC, num_subcores=NS
    )
    return pl.kernel(
        _scatter_body,
        out_type=jax.ShapeDtypeStruct((NC, ACC_ROWS, D), jnp.float32),
        mesh=mesh,
        scratch_types=[
            pltpu.VMEM((EPW_PAD,), jnp.int32),
            pltpu.VMEM((EPW_PAD,), jnp.int32),
            pltpu.VMEM((2, K, D), jnp.float32),
            pltpu.VMEM_SHARED((ACC_ROWS, D), jnp.float32),
            pltpu.SemaphoreType.DMA((2,)),
        ],
    )(hw, srcr, dstr, zeros)


def kernel(t, h, edge_index, norm, weight, bias):
    hw = pl.pallas_call(
        _mm_body,
        out_shape=jax.ShapeDtypeStruct((N, D), jnp.float32),
    )(h, weight, norm)

    src = edge_index[0]
    dst = edge_index[1]
    pad = E_PAD - E
    srcr = jnp.pad(src, (0, pad)).reshape(NW, EPW_PAD)
    dstr = jnp.pad(dst, (0, pad), constant_values=N).reshape(NW, EPW_PAD)
    zeros = jnp.zeros((ROWS_PER_TILE, D), jnp.float32)

    parts = _scatter_call(hw, srcr, dstr, zeros)[:, :N, :]

    return pl.pallas_call(
        _epilogue_body,
        out_shape=jax.ShapeDtypeStruct((N, D), jnp.float32),
    )(parts, norm, bias, t.reshape(1, 1))
